# baseline (speedup 1.0000x reference)
import functools

import jax
import jax.numpy as jnp
from jax import lax
from jax.experimental import pallas as pl
from jax.experimental.pallas import tpu as pltpu
from jax.experimental.pallas import tpu_sc as plsc

B = 4096
F = 26
V = 100000
D = 16
NT = F * V

NC, NS = 2, 16
NW = NC * NS
BPW = B // NW
IPW = BPW * F
GCH = 128
NCHUNK = IPW // GCH

_mesh = plsc.VectorSubcoreMesh(core_axis_name="c", subcore_axis_name="s")


@functools.partial(
    pl.kernel,
    out_type=jax.ShapeDtypeStruct((B,), jnp.float32),
    mesh=_mesh,
    scratch_types=[
        pltpu.VMEM((IPW,), jnp.int32),
        pltpu.VMEM((D * IPW,), jnp.float32),
        pltpu.VMEM((IPW + D,), jnp.float32),
        pltpu.VMEM((BPW,), jnp.float32),
        pltpu.VMEM((16,), jnp.float32),
        pltpu.SemaphoreType.DMA,
        pltpu.SemaphoreType.DMA,
    ],
    compiler_params=pltpu.CompilerParams(
        needs_layout_passes=False, use_tc_tiling_on_sc=False),
)
def _fm_fwd(idx_hbm, wide_hbm, embt_hbm, bias_hbm, out_hbm,
            idx_v, cols_v, wv_v, acc_v, bias_v, sem_e, sem_w):
    wid = lax.axis_index("s") * NC + lax.axis_index("c")
    base = wid * BPW

    pltpu.sync_copy(idx_hbm.at[pl.ds(base * F, IPW)], idx_v)
    pltpu.sync_copy(bias_hbm, bias_v)

    lanes = lax.iota(jnp.int32, 16)

    def chunk_body(c, carry):
        cbase = c * GCH
        for v in range(GCH // 16):
            off = cbase + v * 16
            sl = pl.ds(off, 16)
            pos = lanes + off
            fld = lax.rem(pos, F)
            idx_v[sl] = idx_v[sl] + fld * V
        csl = pl.ds(cbase, GCH)
        for d in range(D):
            pltpu.async_copy(
                embt_hbm.at[d].at[idx_v.at[csl]],
                cols_v.at[pl.ds(d * IPW + cbase, GCH)], sem_e)
        pltpu.async_copy(wide_hbm.at[idx_v.at[csl]],
                         wv_v.at[pl.ds(cbase, GCH)], sem_w)
        return carry

    lax.fori_loop(0, NCHUNK, chunk_body, 0)

    pltpu.make_async_copy(
        embt_hbm.at[0].at[pl.ds(0, D * IPW)], cols_v, sem_e).wait()
    pltpu.make_async_copy(
        wide_hbm.at[pl.ds(0, IPW)], wv_v.at[pl.ds(0, IPW)], sem_w).wait()

    bias_vec = bias_v[...]
    lane26 = lanes * F

    def grp_body(g, carry):
        ib = g * (16 * F)
        eidx0 = lane26 + ib
        w_acc = plsc.load_gather(wv_v, [eidx0])
        for f in range(1, F):
            w_acc = w_acc + plsc.load_gather(wv_v, [eidx0 + f])
        s_list = []
        q_acc = None
        for d in range(D):
            ei = eidx0 + d * IPW
            s_d = plsc.load_gather(cols_v, [ei])
            q_d = s_d * s_d
            for f in range(1, F):
                e = plsc.load_gather(cols_v, [ei + f])
                s_d = s_d + e
                q_d = q_d + e * e
            s_list.append(s_d)
            q_acc = q_d if q_acc is None else q_acc + q_d
        fm = s_list[0] * s_list[0]
        for d in range(1, D):
            fm = fm + s_list[d] * s_list[d]
        x = w_acc + 0.5 * (fm - q_acc) + bias_vec
        acc_v[pl.ds(g * 16, 16)] = 1.0 / (1.0 + jnp.exp(-x))
        return carry

    lax.fori_loop(0, BPW // 16, grp_body, 0)

    pltpu.sync_copy(acc_v, out_hbm.at[pl.ds(base, BPW)])


def kernel(indices, wide_table, emb_table, bias):
    flat_ids = indices.reshape(B * F)
    embt = emb_table.T
    bias16 = jnp.broadcast_to(bias, (16,))
    out = _fm_fwd(flat_ids, wide_table, embt, bias16)
    return out.reshape(B, 1)


# speedup vs baseline: 2.8836x; 2.8836x over previous
import functools

import jax
import jax.numpy as jnp
from jax import lax
from jax.experimental import pallas as pl
from jax.experimental.pallas import tpu as pltpu
from jax.experimental.pallas import tpu_sc as plsc

B = 4096
F = 26
V = 100000
D = 16

NC, NS = 2, 16
NW = NC * NS
BPW = B // NW
IPW = BPW * F
GCH = 128
NCHUNK = IPW // GCH

_mesh = plsc.VectorSubcoreMesh(core_axis_name="c", subcore_axis_name="s")


@functools.partial(
    pl.kernel,
    out_type=jax.ShapeDtypeStruct((B,), jnp.float32),
    mesh=_mesh,
    scratch_types=[
        pltpu.VMEM((IPW,), jnp.int32),
        pltpu.VMEM((IPW, D), jnp.float32),
        pltpu.VMEM((IPW + D,), jnp.float32),
        pltpu.VMEM((BPW,), jnp.float32),
        pltpu.VMEM((16,), jnp.float32),
        pltpu.SemaphoreType.DMA,
        pltpu.SemaphoreType.DMA,
    ],
    compiler_params=pltpu.CompilerParams(
        needs_layout_passes=False, use_tc_tiling_on_sc=False),
)
def _fm_fwd(idx_hbm, wide_hbm, emb_hbm, bias_hbm, out_hbm,
            idx_v, rows_v, wv_v, acc_v, bias_v, sem_e, sem_w):
    wid = lax.axis_index("s") * NC + lax.axis_index("c")
    base = wid * BPW

    pltpu.sync_copy(idx_hbm.at[pl.ds(base * F, IPW)], idx_v)
    pltpu.sync_copy(bias_hbm, bias_v)

    lanes = lax.iota(jnp.int32, 16)

    def chunk_body(c, carry):
        cbase = c * GCH
        for v in range(GCH // 16):
            off = cbase + v * 16
            sl = pl.ds(off, 16)
            pos = lanes + off
            fld = lax.rem(pos, F)
            idx_v[sl] = idx_v[sl] + fld * V
        csl = pl.ds(cbase, GCH)
        pltpu.async_copy(emb_hbm.at[idx_v.at[csl]], rows_v.at[csl, :], sem_e)
        pltpu.async_copy(wide_hbm.at[idx_v.at[csl]], wv_v.at[csl], sem_w)
        return carry

    lax.fori_loop(0, NCHUNK, chunk_body, 0)

    pltpu.make_async_copy(
        emb_hbm.at[pl.ds(0, IPW), :], rows_v, sem_e).wait()
    pltpu.make_async_copy(
        wide_hbm.at[pl.ds(0, IPW)], wv_v.at[pl.ds(0, IPW)], sem_w).wait()

    mask10 = jnp.where(lanes < (F - 16), 1.0, 0.0).astype(jnp.float32)
    bias_vec = bias_v[...]
    zero16 = jnp.zeros((16,), jnp.float32)

    def grp_body(g, carry):
        gb = g * 16
        acc = zero16
        for l in range(16):
            rb = (gb + l) * F
            e0 = rows_v[rb, :]
            s = e0
            q = e0 * e0
            for f in range(1, F):
                e = rows_v[rb + f, :]
                s = s + e
                q = q + e * e
            fmv = s * s - q
            w1 = wv_v[pl.ds(rb, 16)]
            w2 = wv_v[pl.ds(rb + 16, 16)]
            t = 0.5 * fmv + w1 + w2 * mask10
            acc = jnp.where(lanes == l, jnp.sum(t), acc)
        x = acc + bias_vec
        acc_v[pl.ds(gb, 16)] = 1.0 / (1.0 + jnp.exp(-x))
        return carry

    lax.fori_loop(0, BPW // 16, grp_body, 0)

    pltpu.sync_copy(acc_v, out_hbm.at[pl.ds(base, BPW)])


def kernel(indices, wide_table, emb_table, bias):
    flat_ids = indices.reshape(B * F)
    bias16 = jnp.broadcast_to(bias, (16,))
    out = _fm_fwd(flat_ids, wide_table, emb_table, bias16)
    return out.reshape(B, 1)
